# baseline (speedup 1.0000x reference)
import jax
import jax.numpy as jnp
from jax import lax
from jax.experimental import pallas as pl
from jax.experimental.pallas import tpu as pltpu
from jax.experimental.pallas import tpu_sc as plsc

NROWS = 1000000
EMBED_DIM = 32
MLP_HIDDEN = 64
BATCH = 16384

NC = 2
NS = 16
NW = NC * NS
BPW = BATCH // NW
K = 32
DEPTH = 256
UNROLL = 8


S_HEAD = 4096
SPT = S_HEAD // NW


def _sc_gather_body(ii_hbm, ui_hbm, iemb_hbm, uemb_hbm, irows_hbm, uhead_hbm,
                    idx_i, idx_u, sem):
    wid = lax.axis_index("s") * NC + lax.axis_index("c")
    base = wid * BPW
    ubase = wid * SPT
    pltpu.sync_copy(ii_hbm.at[pl.ds(base, BPW)], idx_i)
    pltpu.sync_copy(ui_hbm.at[pl.ds(ubase, SPT)], idx_u)

    def chunk(c, carry):
        co = c * K
        waits = []
        for v in range(K // 16):
            vec = idx_i[pl.ds(co + v * 16, 16)]
            for b in range(16):
                i = co + v * 16 + b
                waits.append(pltpu.async_copy(
                    iemb_hbm.at[pl.ds(vec[b], 1)],
                    irows_hbm.at[pl.ds(base + i, 1)], sem))
        for w in waits:
            w.wait()
        return carry

    lax.fori_loop(0, BPW // K, chunk, 0)

    def uchunk(c, carry):
        co = c * K
        waits = []
        for v in range(K // 16):
            vec = idx_u[pl.ds(co + v * 16, 16)]
            for b in range(16):
                i = co + v * 16 + b
                waits.append(pltpu.async_copy(
                    uemb_hbm.at[pl.ds(vec[b], 1)],
                    uhead_hbm.at[pl.ds(ubase + i, 1)], sem))
        for w in waits:
            w.wait()
        return carry

    lax.fori_loop(0, SPT // K, uchunk, 0)


def _tc_gather_mlp_body(idx_ref, tab_ref, uhead_ref, v_ref, w1t_ref, b1_ref,
                        w2_ref, b2_ref, out_ref, u_vmem, sem, hsem):
    hcopy = pltpu.make_async_copy(
        uhead_ref, u_vmem.at[pl.ds(0, S_HEAD)], hsem)
    hcopy.start()

    def prologue(j, carry):
        for u in range(UNROLL):
            i = S_HEAD + j * UNROLL + u
            pltpu.make_async_copy(
                tab_ref.at[pl.ds(idx_ref[i], 1)], u_vmem.at[pl.ds(i, 1)],
                sem.at[lax.bitwise_and(i, DEPTH - 1)]).start()
        return carry

    lax.fori_loop(0, DEPTH // UNROLL, prologue, 0)

    def body(j, carry):
        for u in range(UNROLL):
            i = S_HEAD + DEPTH + j * UNROLL + u
            s = lax.bitwise_and(i, DEPTH - 1)
            pltpu.make_async_copy(
                tab_ref.at[pl.ds(0, 1)], u_vmem.at[pl.ds(0, 1)],
                sem.at[s]).wait()
            pltpu.make_async_copy(
                tab_ref.at[pl.ds(idx_ref[i], 1)], u_vmem.at[pl.ds(i, 1)],
                sem.at[s]).start()
        return carry

    lax.fori_loop(0, (BATCH - S_HEAD - DEPTH) // UNROLL, body, 0)
    for s in range(DEPTH):
        pltpu.make_async_copy(
            tab_ref.at[pl.ds(0, 1)], u_vmem.at[pl.ds(0, 1)],
            sem.at[s]).wait()
    hcopy.wait()

    h = (jnp.dot(u_vmem[...], w1t_ref[:EMBED_DIM, :],
                 preferred_element_type=jnp.float32)
         + jnp.dot(v_ref[...], w1t_ref[EMBED_DIM:, :],
                   preferred_element_type=jnp.float32)
         + b1_ref[...])
    h = jnp.maximum(h, 0.0)
    o = jnp.sum(h * w2_ref[...], axis=1) + b2_ref[0, 0]
    out_ref[...] = jax.nn.sigmoid(o)


def kernel(user_indices, item_indices, user_emb, item_emb, W1, b1, W2, b2):
    ui = user_indices.astype(jnp.int32)
    ii = item_indices.astype(jnp.int32)

    mesh = plsc.VectorSubcoreMesh(core_axis_name="c", subcore_axis_name="s")
    sc_gather = pl.kernel(
        _sc_gather_body,
        mesh=mesh,
        out_type=[
            jax.ShapeDtypeStruct((BATCH, EMBED_DIM), jnp.float32),
            jax.ShapeDtypeStruct((S_HEAD, EMBED_DIM), jnp.float32),
        ],
        scratch_types=[
            pltpu.VMEM((BPW,), jnp.int32),
            pltpu.VMEM((SPT,), jnp.int32),
            pltpu.SemaphoreType.DMA,
        ],
    )
    v_rows, u_head = sc_gather(ii, ui, item_emb, user_emb)

    out = pl.pallas_call(
        _tc_gather_mlp_body,
        grid_spec=pltpu.PrefetchScalarGridSpec(
            num_scalar_prefetch=1,
            grid=(1,),
            in_specs=[
                pl.BlockSpec(memory_space=pltpu.MemorySpace.HBM),
                pl.BlockSpec(memory_space=pltpu.MemorySpace.HBM),
                pl.BlockSpec((BATCH, EMBED_DIM), lambda i, *_: (0, 0)),
                pl.BlockSpec((2 * EMBED_DIM, MLP_HIDDEN), lambda i, *_: (0, 0)),
                pl.BlockSpec((1, MLP_HIDDEN), lambda i, *_: (0, 0)),
                pl.BlockSpec((1, MLP_HIDDEN), lambda i, *_: (0, 0)),
                pl.BlockSpec((1, 1), lambda i, *_: (0, 0)),
            ],
            out_specs=pl.BlockSpec((BATCH,), lambda i, *_: (0,)),
            scratch_shapes=[
                pltpu.VMEM((BATCH, EMBED_DIM), jnp.float32),
                pltpu.SemaphoreType.DMA((DEPTH,)),
                pltpu.SemaphoreType.DMA,
            ],
        ),
        out_shape=jax.ShapeDtypeStruct((BATCH,), jnp.float32),
    )(ui, user_emb, u_head, v_rows, W1.T, b1.reshape(1, MLP_HIDDEN), W2,
      b2.reshape(1, 1))
    return out


# speedup vs baseline: 1.4240x; 1.4240x over previous
import jax
import jax.numpy as jnp
from jax import lax
from jax.experimental import pallas as pl
from jax.experimental.pallas import tpu as pltpu
from jax.experimental.pallas import tpu_sc as plsc

NROWS = 1000000
EMBED_DIM = 32
MLP_HIDDEN = 64
BATCH = 16384

NC = 2
NS = 16
NW = NC * NS
BPW = BATCH // NW
K = 32
DEPTH = 256
UNROLL = 8


def _sc_gather_body(ii_hbm, iemb_hbm, irows_hbm, idx_i, sem):
    wid = lax.axis_index("s") * NC + lax.axis_index("c")
    base = wid * BPW
    pltpu.sync_copy(ii_hbm.at[pl.ds(base, BPW)], idx_i)

    def chunk(c, carry):
        co = c * K
        waits = []
        for v in range(K // 16):
            vec = idx_i[pl.ds(co + v * 16, 16)]
            for b in range(16):
                i = co + v * 16 + b
                waits.append(pltpu.async_copy(
                    iemb_hbm.at[pl.ds(vec[b], 1)],
                    irows_hbm.at[pl.ds(base + i, 1)], sem))
        for w in waits:
            w.wait()
        return carry

    lax.fori_loop(0, BPW // K, chunk, 0)


def _tc_gather_mlp_body(idx_ref, tab_ref, v_ref, w1t_ref, b1_ref, w2_ref,
                        b2_ref, out_ref, u_vmem, sem):
    def prologue(j, carry):
        for u in range(UNROLL):
            i = j * UNROLL + u
            pltpu.make_async_copy(
                tab_ref.at[pl.ds(idx_ref[i], 1)], u_vmem.at[pl.ds(i, 1)],
                sem.at[lax.bitwise_and(i, DEPTH - 1)]).start()
        return carry

    lax.fori_loop(0, DEPTH // UNROLL, prologue, 0)

    def body(j, carry):
        for u in range(UNROLL):
            i = DEPTH + j * UNROLL + u
            s = lax.bitwise_and(i, DEPTH - 1)
            pltpu.make_async_copy(
                tab_ref.at[pl.ds(0, 1)], u_vmem.at[pl.ds(0, 1)],
                sem.at[s]).wait()
            pltpu.make_async_copy(
                tab_ref.at[pl.ds(idx_ref[i], 1)], u_vmem.at[pl.ds(i, 1)],
                sem.at[s]).start()
        return carry

    lax.fori_loop(0, (BATCH - DEPTH) // UNROLL, body, 0)
    for s in range(DEPTH):
        pltpu.make_async_copy(
            tab_ref.at[pl.ds(0, 1)], u_vmem.at[pl.ds(0, 1)],
            sem.at[s]).wait()

    h = (jnp.dot(u_vmem[...], w1t_ref[:EMBED_DIM, :],
                 preferred_element_type=jnp.float32)
         + jnp.dot(v_ref[...], w1t_ref[EMBED_DIM:, :],
                   preferred_element_type=jnp.float32)
         + b1_ref[...])
    h = jnp.maximum(h, 0.0)
    o = jnp.sum(h * w2_ref[...], axis=1) + b2_ref[0, 0]
    out_ref[...] = jax.nn.sigmoid(o)


def kernel(user_indices, item_indices, user_emb, item_emb, W1, b1, W2, b2):
    ui = user_indices.astype(jnp.int32)
    ii = item_indices.astype(jnp.int32)

    mesh = plsc.VectorSubcoreMesh(core_axis_name="c", subcore_axis_name="s")
    sc_gather = pl.kernel(
        _sc_gather_body,
        mesh=mesh,
        out_type=jax.ShapeDtypeStruct((BATCH, EMBED_DIM), jnp.float32),
        scratch_types=[
            pltpu.VMEM((BPW,), jnp.int32),
            pltpu.SemaphoreType.DMA,
        ],
    )
    v_rows = sc_gather(ii, item_emb)

    out = pl.pallas_call(
        _tc_gather_mlp_body,
        grid_spec=pltpu.PrefetchScalarGridSpec(
            num_scalar_prefetch=1,
            grid=(1,),
            in_specs=[
                pl.BlockSpec(memory_space=pltpu.MemorySpace.HBM),
                pl.BlockSpec((BATCH, EMBED_DIM), lambda i, *_: (0, 0)),
                pl.BlockSpec((2 * EMBED_DIM, MLP_HIDDEN), lambda i, *_: (0, 0)),
                pl.BlockSpec((1, MLP_HIDDEN), lambda i, *_: (0, 0)),
                pl.BlockSpec((1, MLP_HIDDEN), lambda i, *_: (0, 0)),
                pl.BlockSpec((1, 1), lambda i, *_: (0, 0)),
            ],
            out_specs=pl.BlockSpec((BATCH,), lambda i, *_: (0,)),
            scratch_shapes=[
                pltpu.VMEM((BATCH, EMBED_DIM), jnp.float32),
                pltpu.SemaphoreType.DMA((DEPTH,)),
            ],
        ),
        out_shape=jax.ShapeDtypeStruct((BATCH,), jnp.float32),
    )(ui, user_emb, v_rows, W1.T, b1.reshape(1, MLP_HIDDEN), W2,
      b2.reshape(1, 1))
    return out
